# baseline (speedup 1.0000x reference)
import functools

import jax
import jax.numpy as jnp
from jax import lax
from jax.experimental import pallas as pl
from jax.experimental.pallas import tpu as pltpu
from jax.experimental.pallas import tpu_sc as plsc

EMBED_DIM = 64
LANES = 128

_info = plsc.get_sparse_core_info()
_NC, _NS = _info.num_cores, _info.num_subcores
_NW = _NC * _NS

_CHUNK = 128
_NBUF = 4


def _embed_kernel(n_rows: int):
  b_per_w = n_rows // _NW
  n_chunks = b_per_w // _CHUNK
  mesh = plsc.VectorSubcoreMesh(core_axis_name="c", subcore_axis_name="s")

  @functools.partial(
      pl.kernel,
      out_type=jax.ShapeDtypeStruct((n_rows, LANES), jnp.float32),
      mesh=mesh,
      scratch_types=(
          [pltpu.VMEM((n_chunks, _CHUNK), jnp.int32)]
          + [pltpu.VMEM((_CHUNK, LANES), jnp.float32) for _ in range(_NBUF)]
          + [pltpu.SemaphoreType.DMA for _ in range(2 * _NBUF)]
      ),
      compiler_params=pltpu.CompilerParams(use_tc_tiling_on_sc=True),
  )
  def body(idx_hbm, table_hbm, out_hbm, idx_v, *bufs_and_sems):
    rows = bufs_and_sems[:_NBUF]
    sem_g = bufs_and_sems[_NBUF:2 * _NBUF]
    sem_s = bufs_and_sems[2 * _NBUF:]
    wid = lax.axis_index("s") * _NC + lax.axis_index("c")
    base = wid * b_per_w

    pltpu.sync_copy(idx_hbm.at[pl.ds(wid * n_chunks, n_chunks)], idx_v)

    def fire_gather(c, b):
      return pltpu.async_copy(table_hbm.at[idx_v.at[c]], rows[b], sem_g[b])

    def fire_store(c, b):
      pltpu.async_copy(rows[b], out_hbm.at[pl.ds(base + c * _CHUNK, _CHUNK)],
                       sem_s[b])

    def wait_store(b):
      pltpu.make_async_copy(rows[b], out_hbm.at[pl.ds(base, _CHUNK)],
                            sem_s[b]).wait()

    for b in range(_NBUF):
      fire_gather(b, b).wait()
      fire_store(b, b)

    def step(i, carry):
      for b in range(_NBUF):
        c = i * _NBUF + b
        wait_store(b)
        fire_gather(c, b).wait()
        fire_store(c, b)
      return carry

    lax.fori_loop(1, n_chunks // _NBUF, step, 0)

    for b in range(_NBUF):
      wait_store(b)

  return body


def kernel(x, table):
  n_img, seq = x.shape
  n_rows = n_img * seq
  flat = x.reshape(n_rows // _CHUNK, _CHUNK).astype(jnp.int32)
  table_p = jnp.pad(table, ((0, 0), (0, LANES - EMBED_DIM)))
  out = _embed_kernel(n_rows)(flat, table_p)
  return out.reshape(n_img, seq, LANES)[:, :, :EMBED_DIM]
